# baseline (speedup 1.0000x reference)
import functools

import jax
import jax.numpy as jnp
from jax import lax
from jax.experimental import pallas as pl
from jax.experimental.pallas import tpu as pltpu
from jax.experimental.pallas import tpu_sc as plsc

_K = 8192
_D = 256
_N = 8192
_T = 256
_NT = _N // _T
_LT = 1024 // _T


def _sc_gather(w_hbm, idx_hbm, out_hbm, idx_v, rows_v, sem):
    nc = plsc.get_sparse_core_info().num_cores
    wid = lax.axis_index("s") * nc + lax.axis_index("c")
    rows = rows_v.shape[0]
    base = wid * rows
    pltpu.sync_copy(idx_hbm.at[pl.ds(base, rows)], idx_v)
    pltpu.async_copy(w_hbm.at[idx_v], rows_v, sem).wait()
    pltpu.sync_copy(rows_v, out_hbm.at[pl.ds(base, rows)])


_JT = _NT // 2


def _vq_body(idxr_ref, x_ref, qf_ref, enc_ref, q_ref, loss_ref, counts_ref):
    j = pl.program_id(1)

    @pl.when(j == 0)
    def _init():
        counts_ref[...] = jnp.zeros_like(counts_ref)
        loss_ref[...] = jnp.zeros_like(loss_ref)

    xt = x_ref[0]
    idxv = idxr_ref[...][:, 0:1]
    iota = jax.lax.broadcasted_iota(jnp.int32, (_T, _K), 1)
    enc = (iota == idxv).astype(jnp.float32)
    enc_ref[...] = enc
    qt = jnp.transpose(qf_ref[...], (1, 0))
    q_ref[0] = qt
    diff = qt - xt
    counts_ref[0] += jnp.sum(enc, axis=0, keepdims=True)
    loss_ref[0] += jnp.full((1, 128), jnp.sum(diff * diff))


def kernel(inputs, W):
    B, D, L = inputs.shape
    flat = jnp.transpose(inputs, (0, 2, 1)).reshape(-1, _D)
    distances = (jnp.sum(flat ** 2, axis=1, keepdims=True)
                 + jnp.sum(W ** 2, axis=1)
                 - 2.0 * (flat @ W.T))
    idx = jnp.argmin(distances, axis=1)
    idxr = jnp.broadcast_to(idx[:, None], (_N, 128))

    info = plsc.get_sparse_core_info()
    n_workers = info.num_cores * info.num_subcores
    rows = _N // n_workers
    mesh = plsc.VectorSubcoreMesh(core_axis_name="c", subcore_axis_name="s")
    q_flat = functools.partial(
        pl.kernel, mesh=mesh,
        out_type=jax.ShapeDtypeStruct((_N, _D), jnp.float32),
        scratch_types=[
            pltpu.VMEM((rows,), jnp.int32),
            pltpu.VMEM((rows, _D), jnp.float32),
            pltpu.SemaphoreType.DMA,
        ],
    )(_sc_gather)(W, idx)

    def _t(c, j):
        return c * _JT + j

    enc, q, loss_p, counts = pl.pallas_call(
        _vq_body,
        grid=(2, _JT),
        in_specs=[
            pl.BlockSpec((_T, 128), lambda c, j: (_t(c, j), 0)),
            pl.BlockSpec((1, _D, _T),
                         lambda c, j: (_t(c, j) // _LT, 0, _t(c, j) % _LT)),
            pl.BlockSpec((_T, _D), lambda c, j: (_t(c, j), 0)),
        ],
        out_specs=[
            pl.BlockSpec((_T, _K), lambda c, j: (_t(c, j), 0)),
            pl.BlockSpec((1, _D, _T),
                         lambda c, j: (_t(c, j) // _LT, 0, _t(c, j) % _LT)),
            pl.BlockSpec((1, 1, 128), lambda c, j: (c, 0, 0)),
            pl.BlockSpec((1, 1, _K), lambda c, j: (c, 0, 0)),
        ],
        out_shape=[
            jax.ShapeDtypeStruct((_N, _K), jnp.float32),
            jax.ShapeDtypeStruct((B, D, L), jnp.float32),
            jax.ShapeDtypeStruct((2, 1, 128), jnp.float32),
            jax.ShapeDtypeStruct((2, 1, _K), jnp.float32),
        ],
        compiler_params=pltpu.CompilerParams(
            dimension_semantics=("parallel", "arbitrary")),
    )(idxr, inputs, q_flat)

    loss = (loss_p[0, 0, 0] + loss_p[1, 0, 0]) * (2.0 / (_N * _D))
    p = (counts[0, 0] + counts[1, 0]) * (1.0 / _N)
    perp = jnp.exp(-jnp.sum(p * jnp.log(p + 1e-10)))
    return (loss, q, perp, enc)
